# baseline (speedup 1.0000x reference)
import functools

import jax
import jax.numpy as jnp
from jax import lax
from jax.experimental import pallas as pl
from jax.experimental.pallas import tpu as pltpu
from jax.experimental.pallas import tpu_sc as plsc

N_NODES_C = 10000
N_EDGES_C = 320000
D_FEAT_C = 128

NUM_CORES = 2
NUM_SUBCORES = 16
NUM_WORKERS = NUM_CORES * NUM_SUBCORES
CHUNK = 128
N_CHUNKS = 79
E_PER_TILE = N_CHUNKS * CHUNK
E_PADDED = NUM_WORKERS * E_PER_TILE
ROWS_PER_TILE = N_NODES_C // NUM_SUBCORES
LANES = 16
D_SLICES = D_FEAT_C // LANES
EDGE_GROUPS = CHUNK // LANES


def _sc_partials(x, row3, val2, col2):
  mesh = plsc.VectorSubcoreMesh(core_axis_name="c", subcore_axis_name="s")

  @functools.partial(
      pl.kernel,
      mesh=mesh,
      out_type=jax.ShapeDtypeStruct((NUM_CORES, N_NODES_C, D_FEAT_C),
                                    jnp.float32),
      scratch_types=[
          pltpu.VMEM((E_PER_TILE,), jnp.int32),
          pltpu.VMEM((1, CHUNK), jnp.int32),
          pltpu.VMEM((1, CHUNK), jnp.int32),
          pltpu.VMEM((CHUNK,), jnp.float32),
          pltpu.VMEM((CHUNK,), jnp.float32),
          pltpu.VMEM((CHUNK, D_FEAT_C), jnp.float32),
          pltpu.VMEM((CHUNK, D_FEAT_C), jnp.float32),
          pltpu.VMEM_SHARED((N_NODES_C, D_FEAT_C), jnp.float32),
          pltpu.SemaphoreType.DMA,
          pltpu.SemaphoreType.DMA,
          pltpu.SemaphoreType.DMA,
          pltpu.SemaphoreType.DMA,
          pltpu.SemaphoreType.DMA,
      ],
  )
  def k(x_hbm, row_hbm, val_hbm, col_hbm, out_hbm,
        colv, rowv0, rowv1, valv0, valv1, rows0, rows1, ypar,
        psem, gsem0, gsem1, ssem0, ssem1):
    c = lax.axis_index("c")
    s = lax.axis_index("s")
    wid = c * NUM_SUBCORES + s

    h_col = pltpu.async_copy(col_hbm.at[wid], colv, psem)

    zero = jnp.zeros((LANES,), jnp.float32)

    def zfill(i, _):
      for j in range(D_SLICES):
        rows0[i, pl.ds(j * LANES, LANES)] = zero
      return 0

    lax.fori_loop(0, CHUNK, zfill, 0)
    zbase = s * ROWS_PER_TILE
    zh = []
    for zi in range(ROWS_PER_TILE // CHUNK):
      zh.append(pltpu.async_copy(
          rows0, ypar.at[pl.ds(zbase + zi * CHUNK, CHUNK)], psem))
    ztail = ROWS_PER_TILE % CHUNK
    zh.append(pltpu.async_copy(
        rows0.at[pl.ds(0, ztail)],
        ypar.at[pl.ds(zbase + ROWS_PER_TILE - ztail, ztail)], psem))
    for h in zh:
      h.wait()
    h_col.wait()
    plsc.subcore_barrier()

    def issue_in(j, rows_buf, rowv_buf, valv_buf, sem):
      pltpu.async_copy(x_hbm.at[colv.at[pl.ds(j * CHUNK, CHUNK)]],
                       rows_buf, sem)
      pltpu.async_copy(row_hbm.at[wid, pl.ds(j, 1)], rowv_buf, sem)
      pltpu.async_copy(val_hbm.at[wid, pl.ds(j * CHUNK, CHUNK)],
                       valv_buf, sem)

    def drain_in(j, rows_buf, rowv_buf, valv_buf, sem):
      pltpu.make_async_copy(x_hbm.at[colv.at[pl.ds(j * CHUNK, CHUNK)]],
                            rows_buf, sem).wait()
      pltpu.make_async_copy(row_hbm.at[wid, pl.ds(j, 1)], rowv_buf,
                            sem).wait()
      pltpu.make_async_copy(val_hbm.at[wid, pl.ds(j * CHUNK, CHUNK)],
                            valv_buf, sem).wait()

    def scale(buf, valv_buf):
      @plsc.parallel_loop(0, EDGE_GROUPS, unroll=2)
      def _scale16(g):
        vv = valv_buf[pl.ds(g * LANES, LANES)]
        for l in range(LANES):
          v = vv[l]
          e = g * LANES + l
          for d in range(D_SLICES):
            sl = pl.ds(d * LANES, LANES)
            buf[e, sl] = buf[e, sl] * v

    def phase(j, cur, rcur, vcur, csem, sc_cur,
              nxt, rnxt, vnxt, nsem, sc_nxt):
      @pl.when(j >= 1)
      def _():
        pltpu.make_async_copy(nxt, ypar.at[rnxt.at[0]], sc_nxt).wait()

      @pl.when(j + 1 < N_CHUNKS)
      def _():
        issue_in(j + 1, nxt, rnxt, vnxt, nsem)

      drain_in(j, cur, rcur, vcur, csem)
      scale(cur, vcur)
      pltpu.async_copy(cur, ypar.at[rcur.at[0]], sc_cur, add=True)

    issue_in(0, rows0, rowv0, valv0, gsem0)

    def body(j, _):
      @pl.when(j % 2 == 0)
      def _():
        phase(j, rows0, rowv0, valv0, gsem0, ssem0,
              rows1, rowv1, valv1, gsem1, ssem1)

      @pl.when(j % 2 == 1)
      def _():
        phase(j, rows1, rowv1, valv1, gsem1, ssem1,
              rows0, rowv0, valv0, gsem0, ssem0)

      return 0

    lax.fori_loop(0, N_CHUNKS, body, 0)
    pltpu.make_async_copy(rows0, ypar.at[rowv0.at[0]], ssem0).wait()
    plsc.subcore_barrier()

    W = 624
    pltpu.sync_copy(ypar.at[pl.ds(s * W, W)],
                    out_hbm.at[c, pl.ds(s * W, W)])

    @pl.when(s == 0)
    def _tail():
      tail = N_NODES_C - NUM_SUBCORES * W
      pltpu.sync_copy(ypar.at[pl.ds(NUM_SUBCORES * W, tail)],
                      out_hbm.at[c, pl.ds(NUM_SUBCORES * W, tail)])

  return k(x, row3, val2, col2)


def _combine(partials):
  def body(p_ref, o_ref):
    o_ref[...] = p_ref[0] + p_ref[1]

  blk = 1000
  return pl.pallas_call(
      body,
      grid=(N_NODES_C // blk,),
      in_specs=[pl.BlockSpec((NUM_CORES, blk, D_FEAT_C),
                             lambda i: (0, i, 0))],
      out_specs=pl.BlockSpec((blk, D_FEAT_C), lambda i: (i, 0)),
      out_shape=jax.ShapeDtypeStruct((N_NODES_C, D_FEAT_C), jnp.float32),
  )(partials)


def kernel(x, edge_index, edge_values):
  pad = E_PADDED - N_EDGES_C
  spread = (jnp.arange(pad, dtype=jnp.int32) * 13) % N_NODES_C
  row = jnp.concatenate([edge_index[0], spread])
  col = jnp.concatenate([edge_index[1], spread])
  val = jnp.concatenate([edge_values, jnp.zeros((pad,), jnp.float32)])
  row3 = row.reshape(NUM_WORKERS, N_CHUNKS, CHUNK)
  val2 = val.reshape(NUM_WORKERS, E_PER_TILE)
  col2 = col.reshape(NUM_WORKERS, E_PER_TILE)
  partials = _sc_partials(x, row3, val2, col2)
  return _combine(partials)


# speedup vs baseline: 1.0034x; 1.0034x over previous
import functools

import jax
import jax.numpy as jnp
from jax import lax
from jax.experimental import pallas as pl
from jax.experimental.pallas import tpu as pltpu
from jax.experimental.pallas import tpu_sc as plsc

N_NODES_C = 10000
N_EDGES_C = 320000
D_FEAT_C = 128

NUM_CORES = 2
NUM_SUBCORES = 16
NUM_WORKERS = NUM_CORES * NUM_SUBCORES
CHUNK = 128
N_CHUNKS = 79
E_PER_TILE = N_CHUNKS * CHUNK
E_PADDED = NUM_WORKERS * E_PER_TILE
ROWS_PER_TILE = N_NODES_C // NUM_SUBCORES
LANES = 16
D_SLICES = D_FEAT_C // LANES
EDGE_GROUPS = CHUNK // LANES


def _sc_partials(x, row3, val2, col2):
  mesh = plsc.VectorSubcoreMesh(core_axis_name="c", subcore_axis_name="s")

  @functools.partial(
      pl.kernel,
      mesh=mesh,
      out_type=jax.ShapeDtypeStruct((NUM_CORES, N_NODES_C, D_FEAT_C),
                                    jnp.float32),
      scratch_types=[
          pltpu.VMEM((E_PER_TILE,), jnp.int32),
          pltpu.VMEM((1, CHUNK), jnp.int32),
          pltpu.VMEM((1, CHUNK), jnp.int32),
          pltpu.VMEM((CHUNK,), jnp.float32),
          pltpu.VMEM((CHUNK,), jnp.float32),
          pltpu.VMEM((CHUNK, D_FEAT_C), jnp.float32),
          pltpu.VMEM((CHUNK, D_FEAT_C), jnp.float32),
          pltpu.VMEM_SHARED((N_NODES_C, D_FEAT_C), jnp.float32),
          pltpu.SemaphoreType.DMA,
          pltpu.SemaphoreType.DMA,
          pltpu.SemaphoreType.DMA,
          pltpu.SemaphoreType.DMA,
          pltpu.SemaphoreType.DMA,
      ],
  )
  def k(x_hbm, row_hbm, val_hbm, col_hbm, out_hbm,
        colv, rowv0, rowv1, valv0, valv1, rows0, rows1, ypar,
        psem, gsem0, gsem1, ssem0, ssem1):
    c = lax.axis_index("c")
    s = lax.axis_index("s")
    wid = c * NUM_SUBCORES + s

    h_col = pltpu.async_copy(col_hbm.at[wid], colv, psem)

    zero = jnp.zeros((LANES,), jnp.float32)

    def zfill(i, _):
      for j in range(D_SLICES):
        rows0[i, pl.ds(j * LANES, LANES)] = zero
      return 0

    lax.fori_loop(0, CHUNK, zfill, 0)
    zbase = s * ROWS_PER_TILE
    zh = []
    for zi in range(ROWS_PER_TILE // CHUNK):
      zh.append(pltpu.async_copy(
          rows0, ypar.at[pl.ds(zbase + zi * CHUNK, CHUNK)], psem))
    ztail = ROWS_PER_TILE % CHUNK
    zh.append(pltpu.async_copy(
        rows0.at[pl.ds(0, ztail)],
        ypar.at[pl.ds(zbase + ROWS_PER_TILE - ztail, ztail)], psem))
    for h in zh:
      h.wait()
    h_col.wait()
    plsc.subcore_barrier()

    def issue_in(j, rows_buf, rowv_buf, valv_buf, sem):
      pltpu.async_copy(x_hbm.at[colv.at[pl.ds(j * CHUNK, CHUNK)]],
                       rows_buf, sem)
      pltpu.async_copy(row_hbm.at[wid, pl.ds(j, 1)], rowv_buf, sem)
      pltpu.async_copy(val_hbm.at[wid, pl.ds(j * CHUNK, CHUNK)],
                       valv_buf, sem)

    def drain_in(j, rows_buf, rowv_buf, valv_buf, sem):
      pltpu.make_async_copy(x_hbm.at[colv.at[pl.ds(j * CHUNK, CHUNK)]],
                            rows_buf, sem).wait()
      pltpu.make_async_copy(row_hbm.at[wid, pl.ds(j, 1)], rowv_buf,
                            sem).wait()
      pltpu.make_async_copy(val_hbm.at[wid, pl.ds(j * CHUNK, CHUNK)],
                            valv_buf, sem).wait()

    def scale(buf, valv_buf):
      @plsc.parallel_loop(0, EDGE_GROUPS)
      def _scale16(g):
        vv = valv_buf[pl.ds(g * LANES, LANES)]
        for l in range(LANES):
          v = vv[l]
          e = g * LANES + l
          for d in range(D_SLICES):
            sl = pl.ds(d * LANES, LANES)
            buf[e, sl] = buf[e, sl] * v

    def phase(j, cur, rcur, vcur, csem, sc_cur,
              nxt, rnxt, vnxt, nsem, sc_nxt):
      @pl.when(j >= 1)
      def _():
        pltpu.make_async_copy(nxt, ypar.at[rnxt.at[0]], sc_nxt).wait()

      @pl.when(j + 1 < N_CHUNKS)
      def _():
        issue_in(j + 1, nxt, rnxt, vnxt, nsem)

      drain_in(j, cur, rcur, vcur, csem)
      scale(cur, vcur)
      pltpu.async_copy(cur, ypar.at[rcur.at[0]], sc_cur, add=True)

    issue_in(0, rows0, rowv0, valv0, gsem0)

    def body(j, _):
      @pl.when(j % 2 == 0)
      def _():
        phase(j, rows0, rowv0, valv0, gsem0, ssem0,
              rows1, rowv1, valv1, gsem1, ssem1)

      @pl.when(j % 2 == 1)
      def _():
        phase(j, rows1, rowv1, valv1, gsem1, ssem1,
              rows0, rowv0, valv0, gsem0, ssem0)

      return 0

    lax.fori_loop(0, N_CHUNKS, body, 0)
    pltpu.make_async_copy(rows0, ypar.at[rowv0.at[0]], ssem0).wait()
    plsc.subcore_barrier()

    W = 624
    pltpu.sync_copy(ypar.at[pl.ds(s * W, W)],
                    out_hbm.at[c, pl.ds(s * W, W)])

    @pl.when(s == 0)
    def _tail():
      tail = N_NODES_C - NUM_SUBCORES * W
      pltpu.sync_copy(ypar.at[pl.ds(NUM_SUBCORES * W, tail)],
                      out_hbm.at[c, pl.ds(NUM_SUBCORES * W, tail)])

  return k(x, row3, val2, col2)


def _combine(partials):
  def body(p_ref, o_ref):
    o_ref[...] = p_ref[0] + p_ref[1]

  blk = 1000
  return pl.pallas_call(
      body,
      grid=(N_NODES_C // blk,),
      in_specs=[pl.BlockSpec((NUM_CORES, blk, D_FEAT_C),
                             lambda i: (0, i, 0))],
      out_specs=pl.BlockSpec((blk, D_FEAT_C), lambda i: (i, 0)),
      out_shape=jax.ShapeDtypeStruct((N_NODES_C, D_FEAT_C), jnp.float32),
  )(partials)


def kernel(x, edge_index, edge_values):
  pad = E_PADDED - N_EDGES_C
  spread = (jnp.arange(pad, dtype=jnp.int32) * 13) % N_NODES_C
  row = jnp.concatenate([edge_index[0], spread])
  col = jnp.concatenate([edge_index[1], spread])
  val = jnp.concatenate([edge_values, jnp.zeros((pad,), jnp.float32)])
  row3 = row.reshape(NUM_WORKERS, N_CHUNKS, CHUNK)
  val2 = val.reshape(NUM_WORKERS, E_PER_TILE)
  col2 = col.reshape(NUM_WORKERS, E_PER_TILE)
  partials = _sc_partials(x, row3, val2, col2)
  return _combine(partials)
